# baseline (speedup 1.0000x reference)
import functools

import jax
import jax.numpy as jnp
from jax import lax
from jax.experimental import pallas as pl
from jax.experimental.pallas import tpu as pltpu
from jax.experimental.pallas import tpu_sc as plsc

NW = 32
NCORE = 2
NSUB = 16
CH = 125


def _sc_mesh():
    return plsc.VectorSubcoreMesh(core_axis_name="c", subcore_axis_name="s")


def _make_hist(n, nc):
    def body(idx4, ones_h, zvec, out, idx_v, ones_v, h0, h1, h2, h3, sem):
        c = lax.axis_index("c")
        s = lax.axis_index("s")
        wid = s * NCORE + c
        pltpu.sync_copy(ones_h, ones_v)

        @pl.when(s == 0)
        def _():
            pltpu.sync_copy(zvec, h0)
            pltpu.sync_copy(zvec, h1)
            pltpu.sync_copy(zvec, h2)
            pltpu.sync_copy(zvec, h3)

        plsc.subcore_barrier()
        for k, hk in enumerate((h0, h1, h2, h3)):
            pltpu.sync_copy(idx4.at[k, wid], idx_v)

            def fire(j, carry, hk=hk):
                pltpu.async_copy(ones_v, hk.at[idx_v.at[j]], sem, add=True)
                return carry

            def drain(j, carry, hk=hk):
                pltpu.make_async_copy(ones_v, hk.at[idx_v.at[j]], sem).wait()
                return carry

            lax.fori_loop(0, nc, fire, 0)
            lax.fori_loop(0, nc, drain, 0)
        plsc.subcore_barrier()

        @pl.when(s == 0)
        def _():
            pltpu.sync_copy(h0, out.at[c, 0])
            pltpu.sync_copy(h1, out.at[c, 1])
            pltpu.sync_copy(h2, out.at[c, 2])
            pltpu.sync_copy(h3, out.at[c, 3])

    return pl.kernel(
        body,
        out_type=jax.ShapeDtypeStruct((NCORE, 4, n), jnp.float32),
        mesh=_sc_mesh(),
        scratch_types=[
            pltpu.VMEM((nc, CH), jnp.int32),
            pltpu.VMEM((CH,), jnp.float32),
            pltpu.VMEM_SHARED((n,), jnp.float32),
            pltpu.VMEM_SHARED((n,), jnp.float32),
            pltpu.VMEM_SHARED((n,), jnp.float32),
            pltpu.VMEM_SHARED((n,), jnp.float32),
            pltpu.SemaphoreType.DMA,
        ],
        compiler_params=pltpu.CompilerParams(use_tc_tiling_on_sc=False),
    )


def _make_prop(n, h, nc, k=8):
    rt = n // NSUB
    ng = nc // k
    assert ng * k == nc and ng >= 2

    def body(hw, srcr, dstr, zeros, out, src_v, dst_v, rows, acc, *sems):
        gsem = sems[:k]
        ssem = sems[k:]
        c = lax.axis_index("c")
        s = lax.axis_index("s")
        wid = s * NCORE + c
        pltpu.sync_copy(srcr.at[wid], src_v)
        pltpu.sync_copy(dstr.at[wid], dst_v)
        r0 = s * rt
        pltpu.sync_copy(zeros.at[pl.ds(r0, rt)], acc.at[pl.ds(r0, rt)])
        plsc.subcore_barrier()

        for b in range(k):
            pltpu.async_copy(hw.at[src_v.at[b]], rows.at[b], gsem[b])

        def group(g, carry):
            for b in range(k):
                j = g * k + b
                pltpu.make_async_copy(hw.at[src_v.at[j]], rows.at[b], gsem[b]).wait()
                pltpu.sync_copy(rows.at[b], acc.at[dst_v.at[j]], add=True)
                pltpu.async_copy(hw.at[src_v.at[j + k]], rows.at[b], gsem[b])
            return carry

        lax.fori_loop(0, ng - 1, group, 0)
        for b in range(k):
            j = (ng - 1) * k + b
            pltpu.make_async_copy(hw.at[src_v.at[j]], rows.at[b], gsem[b]).wait()
            pltpu.sync_copy(rows.at[b], acc.at[dst_v.at[j]], add=True)
        plsc.subcore_barrier()
        pltpu.sync_copy(acc.at[pl.ds(r0, rt)], out.at[c, pl.ds(r0, rt)])

    return pl.kernel(
        body,
        out_type=jax.ShapeDtypeStruct((NCORE, n, h), jnp.float32),
        mesh=_sc_mesh(),
        scratch_types=[
            pltpu.VMEM((nc, CH), jnp.int32),
            pltpu.VMEM((nc, CH), jnp.int32),
            pltpu.VMEM((k, CH, h), jnp.float32),
            pltpu.VMEM_SHARED((n, h), jnp.float32),
        ] + [pltpu.SemaphoreType.DMA] * (2 * k),
        compiler_params=pltpu.CompilerParams(use_tc_tiling_on_sc=False),
    )


def _make_prop_cs(n, h, nc, k=10):
    h2 = h // 2
    rt = n // NSUB
    ng = nc // k
    assert ng * k == nc and ng >= 2

    def body(hwA, hwB, srcr, dstr, zeros, out, src_v, dst_v, rows, acc, *gsem):
        c = lax.axis_index("c")
        s = lax.axis_index("s")
        pltpu.sync_copy(srcr.at[s], src_v)
        pltpu.sync_copy(dstr.at[s], dst_v)
        r0 = s * rt
        pltpu.sync_copy(zeros.at[pl.ds(r0, rt)], acc.at[pl.ds(r0, rt)])
        plsc.subcore_barrier()

        def run(hw):
            for b in range(k):
                pltpu.async_copy(hw.at[src_v.at[b]], rows.at[b], gsem[b])

            def group(g, carry):
                for b in range(k):
                    j = g * k + b
                    pltpu.make_async_copy(hw.at[src_v.at[j]], rows.at[b], gsem[b]).wait()
                    pltpu.sync_copy(rows.at[b], acc.at[dst_v.at[j]], add=True)
                    pltpu.async_copy(hw.at[src_v.at[j + k]], rows.at[b], gsem[b])
                return carry

            lax.fori_loop(0, ng - 1, group, 0)
            for b in range(k):
                j = (ng - 1) * k + b
                pltpu.make_async_copy(hw.at[src_v.at[j]], rows.at[b], gsem[b]).wait()
                pltpu.sync_copy(rows.at[b], acc.at[dst_v.at[j]], add=True)

        @pl.when(c == 0)
        def _():
            run(hwA)

        @pl.when(c == 1)
        def _():
            run(hwB)

        plsc.subcore_barrier()
        pltpu.sync_copy(acc.at[pl.ds(r0, rt)], out.at[c, pl.ds(r0, rt)])

    return pl.kernel(
        body,
        out_type=jax.ShapeDtypeStruct((NCORE, n, h2), jnp.float32),
        mesh=_sc_mesh(),
        scratch_types=[
            pltpu.VMEM((nc, CH), jnp.int32),
            pltpu.VMEM((nc, CH), jnp.int32),
            pltpu.VMEM((k, CH, h2), jnp.float32),
            pltpu.VMEM_SHARED((n, h2), jnp.float32),
        ] + [pltpu.SemaphoreType.DMA] * k,
        compiler_params=pltpu.CompilerParams(use_tc_tiling_on_sc=False),
    )


def _tc_pq_mm(degp, x, w):
    hn = w.shape[1]

    def body(d_ref, x_ref, w_ref, pqt_ref, oa_ref, ob_ref):
        pqt = lax.rsqrt(jnp.transpose(d_ref[0] + d_ref[1]) + 1.0)
        pqt_ref[...] = pqt
        hw = jnp.dot(x_ref[...], w_ref[...], precision="highest",
                     preferred_element_type=jnp.float32) * pqt[:, 0:1]
        oa_ref[...] = hw[:, : hn // 2]
        ob_ref[...] = hw[:, hn // 2 :]

    n = degp.shape[2]
    return pl.pallas_call(
        body,
        out_shape=(
            jax.ShapeDtypeStruct((n, 4), jnp.float32),
            jax.ShapeDtypeStruct((n, hn // 2), jnp.float32),
            jax.ShapeDtypeStruct((n, hn // 2), jnp.float32),
        ),
    )(degp, x, w)


def _tc_relu_mm(parts, q, w, p, split_out=True):
    hn = w.shape[1]

    def body(a_ref, q_ref, w_ref, p_ref, *o_refs):
        hmat = jnp.concatenate([a_ref[0], a_ref[1]], axis=1)
        hact = jnp.maximum(hmat * q_ref[...], 0.0)
        hw = jnp.dot(hact, w_ref[...], precision="highest",
                     preferred_element_type=jnp.float32) * p_ref[...]
        if split_out:
            o_refs[0][...] = hw[:, : hn // 2]
            o_refs[1][...] = hw[:, hn // 2 :]
        else:
            o_refs[0][...] = hw

    n = parts.shape[1]
    if split_out:
        out_shape = (
            jax.ShapeDtypeStruct((n, hn // 2), jnp.float32),
            jax.ShapeDtypeStruct((n, hn // 2), jnp.float32),
        )
    else:
        out_shape = jax.ShapeDtypeStruct((n, hn), jnp.float32)
    return pl.pallas_call(body, out_shape=out_shape)(parts, q, w, p)


def _tc_final(parts, q):
    def body(a_ref, q_ref, o_ref):
        o_ref[...] = (a_ref[0] + a_ref[1]) * q_ref[...]

    return pl.pallas_call(
        body, out_shape=jax.ShapeDtypeStruct(parts.shape[1:], jnp.float32)
    )(parts, q)


def kernel(x, edge_index1, edge_index2, W1, W2, W3, W4):
    n, _ = x.shape
    e = edge_index1.shape[1]
    nc = e // (NW * CH)
    assert nc * NW * CH == e
    np_ = ((n + NSUB * 8 - 1) // (NSUB * 8)) * (NSUB * 8)

    nc2 = e // (NSUB * CH)
    src1 = edge_index1[0].reshape(NW, nc, CH)
    dst1 = edge_index1[1].reshape(NW, nc, CH)
    src2 = edge_index2[0].reshape(NW, nc, CH)
    dst2 = edge_index2[1].reshape(NW, nc, CH)
    src1c = edge_index1[0].reshape(NSUB, nc2, CH)
    dst1c = edge_index1[1].reshape(NSUB, nc2, CH)
    src2c = edge_index2[0].reshape(NSUB, nc2, CH)
    dst2c = edge_index2[1].reshape(NSUB, nc2, CH)
    idx4 = jnp.stack([src1, dst1, src2, dst2])

    xp = jnp.pad(x, ((0, np_ - n), (0, 0)))
    ones_h = jnp.ones((CH,), jnp.float32)
    zvec = jnp.zeros((np_,), jnp.float32)
    z32 = jnp.zeros((np_, 32), jnp.float32)
    z16 = jnp.zeros((np_, 16), jnp.float32)

    degp = _make_hist(np_, nc)(idx4, ones_h, zvec)
    pqt, hw1a, hw1b = _tc_pq_mm(degp, xp, W1)
    p1 = pqt[:, 0:1]
    q1 = pqt[:, 1:2]
    p2 = pqt[:, 2:3]
    q2 = pqt[:, 3:4]

    prop64c = _make_prop_cs(np_, 64, nc2)
    prop32c = _make_prop_cs(np_, 32, nc2)
    prop16 = _make_prop(np_, 16, nc)
    parts = prop64c(hw1a, hw1b, src1c, dst1c, z32)
    hw2a, hw2b = _tc_relu_mm(parts, q1, W2, p1)
    parts = prop32c(hw2a, hw2b, src1c, dst1c, z16)
    hw3a, hw3b = _tc_relu_mm(parts, q1, W3, p2)
    parts = prop32c(hw3a, hw3b, src2c, dst2c, z16)
    hw4 = _tc_relu_mm(parts, q2, W4, p2, split_out=False)
    parts = prop16(hw4, src2, dst2, z16)
    return _tc_final(parts, q2)[:n]


# speedup vs baseline: 1.0548x; 1.0548x over previous
import functools

import jax
import jax.numpy as jnp
from jax import lax
from jax.experimental import pallas as pl
from jax.experimental.pallas import tpu as pltpu
from jax.experimental.pallas import tpu_sc as plsc

NW = 32
NCORE = 2
NSUB = 16
CH = 125


def _sc_mesh():
    return plsc.VectorSubcoreMesh(core_axis_name="c", subcore_axis_name="s")


def _make_hist(n, nc):
    def body(idx4, ones_h, zvec, out, idx_v, ones_v, h0, h1, h2, h3, sem):
        c = lax.axis_index("c")
        s = lax.axis_index("s")
        wid = s * NCORE + c
        pltpu.sync_copy(ones_h, ones_v)

        @pl.when(s == 0)
        def _():
            pltpu.sync_copy(zvec, h0)
            pltpu.sync_copy(zvec, h1)
            pltpu.sync_copy(zvec, h2)
            pltpu.sync_copy(zvec, h3)

        plsc.subcore_barrier()
        for k, hk in enumerate((h0, h1, h2, h3)):
            pltpu.sync_copy(idx4.at[k, wid], idx_v)

            def fire(j, carry, hk=hk):
                pltpu.async_copy(ones_v, hk.at[idx_v.at[j]], sem, add=True)
                return carry

            def drain(j, carry, hk=hk):
                pltpu.make_async_copy(ones_v, hk.at[idx_v.at[j]], sem).wait()
                return carry

            lax.fori_loop(0, nc, fire, 0)
            lax.fori_loop(0, nc, drain, 0)
        plsc.subcore_barrier()

        @pl.when(s == 0)
        def _():
            pltpu.sync_copy(h0, out.at[c, 0])
            pltpu.sync_copy(h1, out.at[c, 1])
            pltpu.sync_copy(h2, out.at[c, 2])
            pltpu.sync_copy(h3, out.at[c, 3])

    return pl.kernel(
        body,
        out_type=jax.ShapeDtypeStruct((NCORE, 4, n), jnp.float32),
        mesh=_sc_mesh(),
        scratch_types=[
            pltpu.VMEM((nc, CH), jnp.int32),
            pltpu.VMEM((CH,), jnp.float32),
            pltpu.VMEM_SHARED((n,), jnp.float32),
            pltpu.VMEM_SHARED((n,), jnp.float32),
            pltpu.VMEM_SHARED((n,), jnp.float32),
            pltpu.VMEM_SHARED((n,), jnp.float32),
            pltpu.SemaphoreType.DMA,
        ],
        compiler_params=pltpu.CompilerParams(use_tc_tiling_on_sc=False),
    )


def _make_prop(n, h, nc, k=8):
    rt = n // NSUB
    ng = nc // k
    assert ng * k == nc and ng >= 2

    def body(hw, srcr, dstr, zeros, out, src_v, dst_v, rows, acc, *gsem):
        c = lax.axis_index("c")
        s = lax.axis_index("s")
        wid = s * NCORE + c
        pltpu.sync_copy(srcr.at[wid], src_v)
        pltpu.sync_copy(dstr.at[wid], dst_v)
        r0 = s * rt
        pltpu.sync_copy(zeros.at[pl.ds(r0, rt)], acc.at[pl.ds(r0, rt)])
        plsc.subcore_barrier()

        for b in range(k):
            pltpu.async_copy(hw.at[src_v.at[b]], rows.at[b], gsem[b])

        def group(g, carry):
            for b in range(k):
                j = g * k + b
                pltpu.make_async_copy(hw.at[src_v.at[j]], rows.at[b], gsem[b]).wait()
                pltpu.sync_copy(rows.at[b], acc.at[dst_v.at[j]], add=True)
                pltpu.async_copy(hw.at[src_v.at[j + k]], rows.at[b], gsem[b])
            return carry

        lax.fori_loop(0, ng - 1, group, 0)
        for b in range(k):
            j = (ng - 1) * k + b
            pltpu.make_async_copy(hw.at[src_v.at[j]], rows.at[b], gsem[b]).wait()
            pltpu.sync_copy(rows.at[b], acc.at[dst_v.at[j]], add=True)
        plsc.subcore_barrier()
        pltpu.sync_copy(acc.at[pl.ds(r0, rt)], out.at[c, pl.ds(r0, rt)])

    return pl.kernel(
        body,
        out_type=jax.ShapeDtypeStruct((NCORE, n, h), jnp.float32),
        mesh=_sc_mesh(),
        scratch_types=[
            pltpu.VMEM((nc, CH), jnp.int32),
            pltpu.VMEM((nc, CH), jnp.int32),
            pltpu.VMEM((k, CH, h), jnp.float32),
            pltpu.VMEM_SHARED((n, h), jnp.float32),
        ] + [pltpu.SemaphoreType.DMA] * k,
        compiler_params=pltpu.CompilerParams(use_tc_tiling_on_sc=False),
    )


def _tc_pq_mm(degp, x, w):
    def body(d_ref, x_ref, w_ref, pqt_ref, o_ref):
        pqt = lax.rsqrt(jnp.transpose(d_ref[0] + d_ref[1]) + 1.0)
        pqt_ref[...] = pqt
        o_ref[...] = jnp.dot(x_ref[...], w_ref[...], precision="highest",
                             preferred_element_type=jnp.float32) * pqt[:, 0:1]

    n = degp.shape[2]
    return pl.pallas_call(
        body,
        out_shape=(
            jax.ShapeDtypeStruct((n, 4), jnp.float32),
            jax.ShapeDtypeStruct((n, w.shape[1]), jnp.float32),
        ),
    )(degp, x, w)


def _tc_relu_mm(parts, q, w, p):
    def body(a_ref, q_ref, w_ref, p_ref, o_ref):
        hact = jnp.maximum((a_ref[0] + a_ref[1]) * q_ref[...], 0.0)
        o_ref[...] = jnp.dot(hact, w_ref[...], precision="highest",
                             preferred_element_type=jnp.float32) * p_ref[...]

    n = parts.shape[1]
    return pl.pallas_call(
        body, out_shape=jax.ShapeDtypeStruct((n, w.shape[1]), jnp.float32)
    )(parts, q, w, p)


def _tc_final(parts, q):
    def body(a_ref, q_ref, o_ref):
        o_ref[...] = (a_ref[0] + a_ref[1]) * q_ref[...]

    return pl.pallas_call(
        body, out_shape=jax.ShapeDtypeStruct(parts.shape[1:], jnp.float32)
    )(parts, q)


def kernel(x, edge_index1, edge_index2, W1, W2, W3, W4):
    n, _ = x.shape
    e = edge_index1.shape[1]
    nc = e // (NW * CH)
    assert nc * NW * CH == e
    np_ = ((n + NSUB * 8 - 1) // (NSUB * 8)) * (NSUB * 8)

    src1 = edge_index1[0].reshape(NW, nc, CH)
    dst1 = edge_index1[1].reshape(NW, nc, CH)
    src2 = edge_index2[0].reshape(NW, nc, CH)
    dst2 = edge_index2[1].reshape(NW, nc, CH)
    idx4 = jnp.stack([src1, dst1, src2, dst2])

    xp = jnp.pad(x, ((0, np_ - n), (0, 0)))
    ones_h = jnp.ones((CH,), jnp.float32)
    zvec = jnp.zeros((np_,), jnp.float32)
    z64 = jnp.zeros((np_, 64), jnp.float32)
    z32 = jnp.zeros((np_, 32), jnp.float32)
    z16 = jnp.zeros((np_, 16), jnp.float32)

    degp = _make_hist(np_, nc)(idx4, ones_h, zvec)
    pqt, hw1 = _tc_pq_mm(degp, xp, W1)
    p1 = pqt[:, 0:1]
    q1 = pqt[:, 1:2]
    p2 = pqt[:, 2:3]
    q2 = pqt[:, 3:4]

    prop64 = _make_prop(np_, 64, nc, k=8)
    prop32 = _make_prop(np_, 32, nc, k=16)
    prop16 = _make_prop(np_, 16, nc, k=16)
    parts = prop64(hw1, src1, dst1, z64)
    hw2 = _tc_relu_mm(parts, q1, W2, p1)
    parts = prop32(hw2, src1, dst1, z32)
    hw3 = _tc_relu_mm(parts, q1, W3, p2)
    parts = prop32(hw3, src2, dst2, z32)
    hw4 = _tc_relu_mm(parts, q2, W4, p2)
    parts = prop16(hw4, src2, dst2, z16)
    return _tc_final(parts, q2)[:n]


# speedup vs baseline: 1.0648x; 1.0095x over previous
import functools

import jax
import jax.numpy as jnp
from jax import lax
from jax.experimental import pallas as pl
from jax.experimental.pallas import tpu as pltpu
from jax.experimental.pallas import tpu_sc as plsc

NW = 32
NCORE = 2
NSUB = 16
CH = 125


def _sc_mesh():
    return plsc.VectorSubcoreMesh(core_axis_name="c", subcore_axis_name="s")


def _make_hist(n, nc):
    def body(idx4, ones_h, zvec, out, idx_v, ones_v, h0, h1, h2, h3, sem):
        c = lax.axis_index("c")
        s = lax.axis_index("s")
        wid = s * NCORE + c
        pltpu.sync_copy(ones_h, ones_v)

        @pl.when(s == 0)
        def _():
            pltpu.sync_copy(zvec, h0)
            pltpu.sync_copy(zvec, h1)
            pltpu.sync_copy(zvec, h2)
            pltpu.sync_copy(zvec, h3)

        plsc.subcore_barrier()
        for k, hk in enumerate((h0, h1, h2, h3)):
            pltpu.sync_copy(idx4.at[k, wid], idx_v)

            def fire(j, carry, hk=hk):
                pltpu.async_copy(ones_v, hk.at[idx_v.at[j]], sem, add=True)
                return carry

            def drain(j, carry, hk=hk):
                pltpu.make_async_copy(ones_v, hk.at[idx_v.at[j]], sem).wait()
                return carry

            lax.fori_loop(0, nc, fire, 0)
            lax.fori_loop(0, nc, drain, 0)
        plsc.subcore_barrier()

        @pl.when(s == 0)
        def _():
            pltpu.sync_copy(h0, out.at[c, 0])
            pltpu.sync_copy(h1, out.at[c, 1])
            pltpu.sync_copy(h2, out.at[c, 2])
            pltpu.sync_copy(h3, out.at[c, 3])

    return pl.kernel(
        body,
        out_type=jax.ShapeDtypeStruct((NCORE, 4, n), jnp.float32),
        mesh=_sc_mesh(),
        scratch_types=[
            pltpu.VMEM((nc, CH), jnp.int32),
            pltpu.VMEM((CH,), jnp.float32),
            pltpu.VMEM_SHARED((n,), jnp.float32),
            pltpu.VMEM_SHARED((n,), jnp.float32),
            pltpu.VMEM_SHARED((n,), jnp.float32),
            pltpu.VMEM_SHARED((n,), jnp.float32),
            pltpu.SemaphoreType.DMA,
        ],
        compiler_params=pltpu.CompilerParams(use_tc_tiling_on_sc=False),
    )


def _make_prop(n, h, nc, k=8):
    rt = n // NSUB
    ng = nc // k
    assert ng * k == nc and ng >= 2

    def body(hw, srcr, dstr, zeros, out, src_v, dst_v, rows, acc, *gsem):
        c = lax.axis_index("c")
        s = lax.axis_index("s")
        wid = s * NCORE + c
        pltpu.sync_copy(srcr.at[wid], src_v)
        pltpu.sync_copy(dstr.at[wid], dst_v)
        r0 = s * rt
        pltpu.sync_copy(zeros.at[pl.ds(r0, rt)], acc.at[pl.ds(r0, rt)])
        plsc.subcore_barrier()

        for b in range(k):
            pltpu.async_copy(hw.at[src_v.at[b]], rows.at[b], gsem[b])

        def group(g, carry):
            for b in range(k):
                j = g * k + b
                pltpu.make_async_copy(hw.at[src_v.at[j]], rows.at[b], gsem[b]).wait()
                pltpu.sync_copy(rows.at[b], acc.at[dst_v.at[j]], add=True)
                pltpu.async_copy(hw.at[src_v.at[j + k]], rows.at[b], gsem[b])
            return carry

        lax.fori_loop(0, ng - 1, group, 0)
        for b in range(k):
            j = (ng - 1) * k + b
            pltpu.make_async_copy(hw.at[src_v.at[j]], rows.at[b], gsem[b]).wait()
            pltpu.sync_copy(rows.at[b], acc.at[dst_v.at[j]], add=True)
        plsc.subcore_barrier()
        pltpu.sync_copy(acc.at[pl.ds(r0, rt)], out.at[c, pl.ds(r0, rt)])

    return pl.kernel(
        body,
        out_type=jax.ShapeDtypeStruct((NCORE, n, h), jnp.float32),
        mesh=_sc_mesh(),
        scratch_types=[
            pltpu.VMEM((nc, CH), jnp.int32),
            pltpu.VMEM((nc, CH), jnp.int32),
            pltpu.VMEM((k, CH, h), jnp.float32),
            pltpu.VMEM_SHARED((n, h), jnp.float32),
        ] + [pltpu.SemaphoreType.DMA] * k,
        compiler_params=pltpu.CompilerParams(use_tc_tiling_on_sc=False),
    )


def _tc_pq_mm(degp, x, w):
    def body(d_ref, x_ref, w_ref, pqt_ref, o_ref):
        pqt = lax.rsqrt(jnp.transpose(d_ref[0] + d_ref[1]) + 1.0)
        pqt_ref[...] = pqt
        o_ref[...] = jnp.dot(x_ref[...], w_ref[...], precision="highest",
                             preferred_element_type=jnp.float32) * pqt[:, 0:1]

    n = degp.shape[2]
    return pl.pallas_call(
        body,
        out_shape=(
            jax.ShapeDtypeStruct((n, 4), jnp.float32),
            jax.ShapeDtypeStruct((n, w.shape[1]), jnp.float32),
        ),
    )(degp, x, w)


def _tc_relu_mm(parts, q, w, p):
    def body(a_ref, q_ref, w_ref, p_ref, o_ref):
        hact = jnp.maximum((a_ref[0] + a_ref[1]) * q_ref[...], 0.0)
        o_ref[...] = jnp.dot(hact, w_ref[...], precision="highest",
                             preferred_element_type=jnp.float32) * p_ref[...]

    n = parts.shape[1]
    return pl.pallas_call(
        body, out_shape=jax.ShapeDtypeStruct((n, w.shape[1]), jnp.float32)
    )(parts, q, w, p)


def _tc_final(parts, q):
    def body(a_ref, q_ref, o_ref):
        o_ref[...] = (a_ref[0] + a_ref[1]) * q_ref[...]

    return pl.pallas_call(
        body, out_shape=jax.ShapeDtypeStruct(parts.shape[1:], jnp.float32)
    )(parts, q)


def kernel(x, edge_index1, edge_index2, W1, W2, W3, W4):
    n, _ = x.shape
    e = edge_index1.shape[1]
    nc = e // (NW * CH)
    assert nc * NW * CH == e
    np_ = ((n + NSUB * 8 - 1) // (NSUB * 8)) * (NSUB * 8)

    src1 = edge_index1[0].reshape(NW, nc, CH)
    dst1 = edge_index1[1].reshape(NW, nc, CH)
    src2 = edge_index2[0].reshape(NW, nc, CH)
    dst2 = edge_index2[1].reshape(NW, nc, CH)
    idx4 = jnp.stack([src1, dst1, src2, dst2])

    xp = jnp.pad(x, ((0, np_ - n), (0, 0)))
    ones_h = jnp.ones((CH,), jnp.float32)
    zvec = jnp.zeros((np_,), jnp.float32)
    z64 = jnp.zeros((np_, 64), jnp.float32)
    z32 = jnp.zeros((np_, 32), jnp.float32)
    z16 = jnp.zeros((np_, 16), jnp.float32)

    degp = _make_hist(np_, nc)(idx4, ones_h, zvec)
    pqt, hw1 = _tc_pq_mm(degp, xp, W1)
    p1 = pqt[:, 0:1]
    q1 = pqt[:, 1:2]
    p2 = pqt[:, 2:3]
    q2 = pqt[:, 3:4]

    prop64 = _make_prop(np_, 64, nc)
    prop32 = _make_prop(np_, 32, nc)
    prop16 = _make_prop(np_, 16, nc)
    parts = prop64(hw1, src1, dst1, z64)
    hw2 = _tc_relu_mm(parts, q1, W2, p1)
    parts = prop32(hw2, src1, dst1, z32)
    hw3 = _tc_relu_mm(parts, q1, W3, p2)
    parts = prop32(hw3, src2, dst2, z32)
    hw4 = _tc_relu_mm(parts, q2, W4, p2)
    parts = prop16(hw4, src2, dst2, z16)
    return _tc_final(parts, q2)[:n]
